# baseline (speedup 1.0000x reference)
import functools

import jax
import jax.numpy as jnp
from jax import lax
from jax.experimental import pallas as pl
from jax.experimental.pallas import tpu as pltpu
from jax.experimental.pallas import tpu_sc as plsc

_B = 16384
_D = 32
_L = 16
_NC = 2
_NS = 16
_NW = _NC * _NS
_BPW = _B // _NW


def _gmf_body(uids, iids, utab, itab, out,
              uidx_v, iidx_v, urows_v, irows_v, usem, isem):
    wid = lax.axis_index("s") * _NC + lax.axis_index("c")
    base = wid * _BPW
    pltpu.sync_copy(uids.at[pl.ds(base, _BPW)], uidx_v)
    pltpu.sync_copy(iids.at[pl.ds(base, _BPW)], iidx_v)
    cu = pltpu.async_copy(utab.at[uidx_v], urows_v, usem)
    ci = pltpu.async_copy(itab.at[iidx_v], irows_v, isem)
    cu.wait()
    ci.wait()

    def mul_row(i, carry):
        for c in range(_D // _L):
            sl = (i, pl.ds(c * _L, _L))
            urows_v[sl] = urows_v[sl] * irows_v[sl]
        return carry

    lax.fori_loop(0, _BPW, mul_row, 0)
    pltpu.sync_copy(urows_v, out.at[pl.ds(base, _BPW)])


_gmf = functools.partial(
    pl.kernel,
    mesh=plsc.VectorSubcoreMesh(core_axis_name="c", subcore_axis_name="s"),
    compiler_params=pltpu.CompilerParams(use_tc_tiling_on_sc=False),
    out_type=jax.ShapeDtypeStruct((_B, _D), jnp.float32),
    scratch_types=[
        pltpu.VMEM((_BPW,), jnp.int32),
        pltpu.VMEM((_BPW,), jnp.int32),
        pltpu.VMEM((_BPW, _D), jnp.float32),
        pltpu.VMEM((_BPW, _D), jnp.float32),
        pltpu.SemaphoreType.DMA,
        pltpu.SemaphoreType.DMA,
    ],
)(_gmf_body)


def kernel(user_ids, item_ids, user_table, item_table):
    return _gmf(user_ids.astype(jnp.int32), item_ids.astype(jnp.int32),
                user_table, item_table)
